# baseline (speedup 1.0000x reference)
import functools

import jax
import jax.numpy as jnp
from jax import lax
from jax.experimental import pallas as pl
from jax.experimental.pallas import tpu as pltpu
from jax.experimental.pallas import tpu_sc as plsc

E = 16
D = 1024
F = 2816
T = 32
BF = 1408
NF = F // BF

_SC_MESH = plsc.VectorSubcoreMesh(core_axis_name="c", subcore_axis_name="s")


@functools.partial(
    pl.kernel,
    mesh=_SC_MESH,
    out_type=[
        jax.ShapeDtypeStruct((T, E), jnp.float32),
        jax.ShapeDtypeStruct((T, E), jnp.int32),
    ],
    scratch_types=[
        pltpu.VMEM((D,), jnp.float32),
        pltpu.VMEM((D * E,), jnp.float32),
        pltpu.VMEM((E,), jnp.float32),
        pltpu.VMEM((E,), jnp.int32),
        pltpu.VMEM((E,), jnp.float32),
    ],
)
def _gating_sc(x_hbm, wsw_hbm, bsw_hbm, wi_hbm, ids_hbm,
               x_v, wsw_v, bsw_v, ids_v, wi_v):
    t = lax.axis_index("s") * 2 + lax.axis_index("c")
    pltpu.sync_copy(x_hbm.at[t], x_v)
    pltpu.sync_copy(wsw_hbm, wsw_v)
    pltpu.sync_copy(bsw_hbm, bsw_v)

    def body(i, acc):
        base = i * 16
        xv = x_v[pl.ds(base, 16)]
        for j in range(16):
            acc = acc + xv[j] * wsw_v[pl.ds((base + j) * E, E)]
        return acc

    logits = lax.fori_loop(0, D // 16, body, jnp.zeros((E,), jnp.float32))
    logits = logits + bsw_v[...]

    lane = jnp.arange(E, dtype=jnp.int32)
    m1 = jnp.float32(-3.0e38)
    m2 = jnp.float32(-3.0e38)
    e_top1 = jnp.int32(0)
    e_top2 = jnp.int32(0)
    for j in range(E):
        v = logits[j]
        nm = v > m1
        n2 = jnp.logical_and(jnp.logical_not(nm), v > m2)
        e_top2 = jnp.where(nm, e_top1, jnp.where(n2, jnp.int32(j), e_top2))
        m2 = jnp.where(nm, m1, jnp.where(n2, v, m2))
        e_top1 = jnp.where(nm, jnp.int32(j), e_top1)
        m1 = jnp.where(nm, v, m1)
    e2 = jnp.exp(jnp.full((E,), m2 - m1, jnp.float32))
    w1 = 1.0 / (1.0 + e2)
    w2 = e2 / (1.0 + e2)
    wi_v[...] = jnp.where(lane == e_top1, w1,
                          jnp.where(lane == e_top2, w2, 0.0))
    ids_v[...] = jnp.where(lane == 0, e_top1, jnp.where(lane == 1, e_top2, 0))
    pltpu.sync_copy(wi_v, wi_hbm.at[t])
    pltpu.sync_copy(ids_v, ids_hbm.at[t])


def _experts_kernel(x_ref, wi_ref, wg_ref, wu_ref, wd_ref, out_ref):
    e = pl.program_id(0)
    f = pl.program_id(1)
    x = x_ref[...]
    g = jnp.dot(x, wg_ref[0], preferred_element_type=jnp.float32)
    g = g * jax.nn.sigmoid(g)
    u = jnp.dot(x, wu_ref[0], preferred_element_type=jnp.float32)
    contrib = jnp.dot(g * u, wd_ref[0], preferred_element_type=jnp.float32)

    lane = jax.lax.broadcasted_iota(jnp.int32, (T, E), 1)
    w_e = jnp.sum(jnp.where(lane == e, wi_ref[...], 0.0), axis=1,
                  keepdims=True)

    @pl.when(jnp.logical_and(e == 0, f == 0))
    def _first():
        out_ref[...] = w_e * contrib

    @pl.when(jnp.logical_or(e != 0, f != 0))
    def _rest():
        out_ref[...] += w_e * contrib


@jax.jit
def kernel(x, W_gate, W_up, W_down, W_switch, b_switch):
    b, n, d = x.shape
    xf = x.reshape(T, d)

    wi, _ids = _gating_sc(xf, W_switch.reshape(D * E), b_switch)

    out = pl.pallas_call(
        _experts_kernel,
        grid=(E, NF),
        in_specs=[
            pl.BlockSpec((T, D), lambda e, f: (0, 0)),
            pl.BlockSpec((T, E), lambda e, f: (0, 0)),
            pl.BlockSpec((1, D, BF), lambda e, f: (e, 0, f)),
            pl.BlockSpec((1, D, BF), lambda e, f: (e, 0, f)),
            pl.BlockSpec((1, BF, D), lambda e, f: (e, f, 0)),
        ],
        out_specs=pl.BlockSpec((T, D), lambda e, f: (0, 0)),
        out_shape=jax.ShapeDtypeStruct((T, D), jnp.float32),
        compiler_params=pltpu.CompilerParams(
            dimension_semantics=("arbitrary", "arbitrary"),
        ),
    )(xf, wi, W_gate, W_up, W_down)
    return out.reshape(b, n, d)


# speedup vs baseline: 1.0429x; 1.0429x over previous
import functools

import jax
import jax.numpy as jnp
from jax import lax
from jax.experimental import pallas as pl
from jax.experimental.pallas import tpu as pltpu
from jax.experimental.pallas import tpu_sc as plsc

E = 16
D = 1024
F = 2816
T = 32
BF = 1408
NF = F // BF

_SC_MESH = plsc.VectorSubcoreMesh(core_axis_name="c", subcore_axis_name="s",
                                  num_cores=1)


@functools.partial(
    pl.kernel,
    mesh=_SC_MESH,
    out_type=[
        jax.ShapeDtypeStruct((T, E), jnp.float32),
        jax.ShapeDtypeStruct((T, E), jnp.int32),
    ],
    scratch_types=[
        pltpu.VMEM((D,), jnp.float32),
        pltpu.VMEM((D * E,), jnp.float32),
        pltpu.VMEM((E,), jnp.float32),
        pltpu.VMEM((E,), jnp.int32),
        pltpu.VMEM((E,), jnp.float32),
    ],
)
def _gating_sc(x_hbm, wsw_hbm, bsw_hbm, wi_hbm, ids_hbm,
               x_v, wsw_v, bsw_v, ids_v, wi_v):
    s = lax.axis_index("s")
    pltpu.sync_copy(wsw_hbm, wsw_v)
    pltpu.sync_copy(bsw_hbm, bsw_v)
    lane = jnp.arange(E, dtype=jnp.int32)

    for k in range(2):
        t = s * 2 + k
        pltpu.sync_copy(x_hbm.at[t], x_v)

        def body(i, acc):
            base = i * 16
            xv = x_v[pl.ds(base, 16)]
            for j in range(16):
                acc = acc + xv[j] * wsw_v[pl.ds((base + j) * E, E)]
            return acc

        logits = lax.fori_loop(0, D // 16, body, jnp.zeros((E,), jnp.float32))
        logits = logits + bsw_v[...]

        m1 = jnp.float32(-3.0e38)
        m2 = jnp.float32(-3.0e38)
        e_top1 = jnp.int32(0)
        e_top2 = jnp.int32(0)
        for j in range(E):
            v = logits[j]
            nm = v > m1
            n2 = jnp.logical_and(jnp.logical_not(nm), v > m2)
            e_top2 = jnp.where(nm, e_top1, jnp.where(n2, jnp.int32(j), e_top2))
            m2 = jnp.where(nm, m1, jnp.where(n2, v, m2))
            e_top1 = jnp.where(nm, jnp.int32(j), e_top1)
            m1 = jnp.where(nm, v, m1)
        e2 = jnp.exp(jnp.full((E,), m2 - m1, jnp.float32))
        w1 = 1.0 / (1.0 + e2)
        w2 = e2 / (1.0 + e2)
        wi_v[...] = jnp.where(lane == e_top1, w1,
                              jnp.where(lane == e_top2, w2, 0.0))
        ids_v[...] = jnp.where(lane == 0, e_top1,
                               jnp.where(lane == 1, e_top2, 0))
        pltpu.sync_copy(wi_v, wi_hbm.at[t])
        pltpu.sync_copy(ids_v, ids_hbm.at[t])


def _experts_kernel(x_ref, wg_ref, wu_ref, wd_ref, out_ref):
    f = pl.program_id(1)
    x = x_ref[...]
    g = jnp.dot(x, wg_ref[0], preferred_element_type=jnp.float32)
    g = g * jax.nn.sigmoid(g)
    u = jnp.dot(x, wu_ref[0], preferred_element_type=jnp.float32)
    contrib = jnp.dot(g * u, wd_ref[0], preferred_element_type=jnp.float32)

    @pl.when(f == 0)
    def _first():
        out_ref[0] = contrib

    @pl.when(f != 0)
    def _rest():
        out_ref[0] += contrib


def _combine_kernel(eo_ref, wi_ref, out_ref):
    acc = jnp.zeros((T, D), jnp.float32)
    for e in range(E):
        acc = acc + wi_ref[:, e:e + 1] * eo_ref[e]
    out_ref[...] = acc


@jax.jit
def kernel(x, W_gate, W_up, W_down, W_switch, b_switch):
    b, n, d = x.shape
    xf = x.reshape(T, d)

    wi, _ids = _gating_sc(xf, W_switch.reshape(D * E), b_switch)

    eo = pl.pallas_call(
        _experts_kernel,
        grid=(E, NF),
        in_specs=[
            pl.BlockSpec((T, D), lambda e, f: (0, 0)),
            pl.BlockSpec((1, D, BF), lambda e, f: (e, 0, f)),
            pl.BlockSpec((1, D, BF), lambda e, f: (e, 0, f)),
            pl.BlockSpec((1, BF, D), lambda e, f: (e, f, 0)),
        ],
        out_specs=pl.BlockSpec((1, T, D), lambda e, f: (e, 0, 0)),
        out_shape=jax.ShapeDtypeStruct((E, T, D), jnp.float32),
        compiler_params=pltpu.CompilerParams(
            dimension_semantics=("arbitrary", "arbitrary"),
        ),
    )(xf, W_gate, W_up, W_down)

    out = pl.pallas_call(
        _combine_kernel,
        in_specs=[
            pl.BlockSpec((E, T, D), lambda: (0, 0, 0)),
            pl.BlockSpec((T, E), lambda: (0, 0)),
        ],
        out_specs=pl.BlockSpec((T, D), lambda: (0, 0)),
        out_shape=jax.ShapeDtypeStruct((T, D), jnp.float32),
    )(eo, wi)
    return out.reshape(b, n, d)
